# baseline (speedup 1.0000x reference)
import functools
import jax
import jax.numpy as jnp
from jax import lax
from jax.experimental import pallas as pl
from jax.experimental.pallas import tpu as pltpu
from jax.experimental.pallas import tpu_sc as plsc


_B, _L, _D = 4, 8192, 1024
_NC, _NS = 2, 16
_NW = _NC * _NS
_RPW = _L // _NW
_CHUNK = 64
_NCHUNK = _RPW // _CHUNK

_mesh = plsc.VectorSubcoreMesh(core_axis_name="c", subcore_axis_name="s")


@functools.partial(
    pl.kernel,
    mesh=_mesh,
    out_type=jax.ShapeDtypeStruct((_B, _L, _D), jnp.float32),
    scratch_types=[pltpu.VMEM((_CHUNK, _D), jnp.float32)],
)
def _sc_broadcast(table_hbm, out_hbm, buf):
    wid = lax.axis_index("s") * _NC + lax.axis_index("c")
    for c in range(_NCHUNK):
        base = wid * _RPW + c * _CHUNK
        pltpu.sync_copy(table_hbm.at[pl.ds(base, _CHUNK)], buf)
        for b in range(_B):
            pltpu.sync_copy(buf, out_hbm.at[b, pl.ds(base, _CHUNK)])


def kernel(x, table):
    return _sc_broadcast(table)


# speedup vs baseline: 1.4278x; 1.4278x over previous
import jax
import jax.numpy as jnp
from jax.experimental import pallas as pl


_BL = 1024


def _copy_body(t_ref, o_ref):
    for b in range(4):
        o_ref[b] = t_ref[...]


def kernel(x, table):
    B, L, D = x.shape
    n_l = L // _BL
    out = pl.pallas_call(
        _copy_body,
        grid=(n_l,),
        in_specs=[pl.BlockSpec((_BL, D), lambda l: (l, 0))],
        out_specs=pl.BlockSpec((B, _BL, D), lambda l: (0, l, 0)),
        out_shape=jax.ShapeDtypeStruct((B, L, D), table.dtype),
    )(table)
    return out
